# Initial kernel scaffold; baseline (speedup 1.0000x reference)
import jax
import jax.numpy as jnp
from jax.experimental import pallas as pl


def kernel(x, weight):
    raise NotImplementedError("write your pallas kernel here")


# speedup vs baseline: 1.4570x; 1.4570x over previous
import jax
import jax.numpy as jnp
from jax.experimental import pallas as pl
from jax.experimental.pallas import tpu as pltpu

_TILE = 64


def _gather_kernel(idx_ref, w_ref, o_ref):
    base = pl.program_id(0) * _TILE
    for mi in range(_TILE):
        o_ref[mi, 0] = w_ref[idx_ref[base + mi], 0]


def _round_up(n, m):
    return ((n + m - 1) // m) * m


def kernel(x, weight):
    B, S = x.shape
    V, D = weight.shape
    N = B * S

    D_pad = _round_up(D, 128)
    if D_pad != D:
        weight = jnp.pad(weight, ((0, 0), (0, D_pad - D)))

    idx = jnp.clip(x.reshape(N).astype(jnp.int32), 0, V - 1)
    N_pad = _round_up(N, _TILE)
    if N_pad != N:
        idx = jnp.pad(idx, (0, N_pad - N))

    out = pl.pallas_call(
        _gather_kernel,
        out_shape=jax.ShapeDtypeStruct((N_pad, 1, D_pad), weight.dtype),
        grid_spec=pltpu.PrefetchScalarGridSpec(
            num_scalar_prefetch=1,
            grid=(N_pad // _TILE,),
            in_specs=[
                pl.BlockSpec((V, 1, D_pad), lambda i, ids: (0, 0, 0)),
            ],
            out_specs=pl.BlockSpec((_TILE, 1, D_pad), lambda i, ids: (i, 0, 0)),
        ),
        compiler_params=pltpu.CompilerParams(
            dimension_semantics=("parallel",),
            vmem_limit_bytes=48 * 1024 * 1024,
        ),
    )(idx, weight.reshape(V, 1, D_pad))

    return out[:N, 0, :D].reshape(B, S, D)
